# baseline (speedup 1.0000x reference)
import jax
import jax.numpy as jnp
from jax.experimental import pallas as pl
from jax.experimental.pallas import tpu as pltpu

B, C, N = 8, 64, 4096
M = N
KA = C + 4


def _hilo(v):
    hi = v.astype(jnp.bfloat16)
    lo = (v - hi.astype(jnp.float32)).astype(jnp.bfloat16)
    return hi, lo


def _sq_colsum(vb):
    vf = vb.astype(jnp.float32)
    return jnp.sum(vf * vf, axis=0, keepdims=True)


def _chamfer_kernel(x_ref, y_ref, out_ref, xa_ref, ya_ref):
    b = pl.program_id(0)

    @pl.when(b == 0)
    def _init():
        out_ref[0, 0] = 0.0
        xa_ref[C:C + 2, :] = jnp.ones((2, N), jnp.bfloat16)
        ya_ref[KA - 2:KA, :] = jnp.ones((2, M), jnp.bfloat16)

    xv = x_ref[0].astype(jnp.bfloat16)
    xa_ref[0:C, :] = xv
    x2_hi, x2_lo = _hilo(_sq_colsum(xv))
    xa_ref[C + 2:C + 3, :] = x2_hi
    xa_ref[C + 3:C + 4, :] = x2_lo
    yv = y_ref[0].astype(jnp.bfloat16)
    ya_ref[0:C, :] = yv * jnp.bfloat16(-2.0)
    y2_hi, y2_lo = _hilo(_sq_colsum(yv))
    ya_ref[C:C + 1, :] = y2_hi
    ya_ref[C + 1:C + 2, :] = y2_lo

    d = jax.lax.dot_general(
        xa_ref[...], ya_ref[...], (((0,), (0,)), ((), ())),
        preferred_element_type=jnp.float32)

    rows = [d[k * 512:(k + 1) * 512, :] for k in range(N // 512)]
    while len(rows) > 1:
        rows = [jnp.minimum(rows[i], rows[i + 1])
                for i in range(0, len(rows), 2)]
    col_min = jnp.maximum(jnp.min(rows[0], axis=0, keepdims=True), 0.0)
    out_ref[0, 0] += (10.0 / (B * M)) * jnp.sum(col_min)

    chunks = [d[:, k * 128:(k + 1) * 128] for k in range(M // 128)]
    while len(chunks) > 1:
        chunks = [jnp.minimum(chunks[i], chunks[i + 1])
                  for i in range(0, len(chunks), 2)]
    rm = jnp.maximum(jnp.min(chunks[0], axis=1, keepdims=True), 0.0)
    out_ref[0, 0] += (1.0 / (B * N)) * jnp.sum(rm)


@jax.jit
def kernel(x, y):
    out = pl.pallas_call(
        _chamfer_kernel,
        grid=(B,),
        in_specs=[
            pl.BlockSpec((1, C, N), lambda b: (b, 0, 0)),
            pl.BlockSpec((1, C, M), lambda b: (b, 0, 0)),
        ],
        out_specs=pl.BlockSpec(memory_space=pltpu.MemorySpace.SMEM),
        out_shape=jax.ShapeDtypeStruct((1, 1), jnp.float32),
        scratch_shapes=[
            pltpu.VMEM((KA, N), jnp.bfloat16),
            pltpu.VMEM((KA, M), jnp.bfloat16),
        ],
    )(x, y)
    return out[0, 0]
